# baseline (speedup 1.0000x reference)
import jax
import jax.numpy as jnp
from jax.experimental import pallas as pl

_K = 128
_ITERS = 10
_CH = 2000


def _dist_idx(cn_ref, ct_ref):
    px = cn_ref[:, 0:1]
    py = cn_ref[:, 1:2]
    pz = cn_ref[:, 2:3]
    cx = ct_ref[0:1, :]
    cy = ct_ref[1:2, :]
    cz = ct_ref[2:3, :]
    dx = px - cx
    dy = py - cy
    dz = pz - cz
    d = (dx * dx + dy * dy) + dz * dz
    m = jnp.min(d, axis=1, keepdims=True)
    ii = jax.lax.broadcasted_iota(jnp.int32, d.shape, 1)
    idx = jnp.min(jnp.where(d == m, ii, jnp.int32(2147483647)), axis=1,
                  keepdims=True)
    return idx


def _argmin_body(cn_ref, ct_ref, lab_ref):
    lab_ref[...] = _dist_idx(cn_ref, ct_ref)


def _labels_pallas(cn, ct):
    N = cn.shape[0]
    out = pl.pallas_call(
        _argmin_body,
        grid=(N // _CH,),
        in_specs=[
            pl.BlockSpec((_CH, 3), lambda i: (i, 0)),
            pl.BlockSpec((3, _K), lambda i: (0, 0)),
        ],
        out_specs=pl.BlockSpec((_CH, 1), lambda i: (i, 0)),
        out_shape=jax.ShapeDtypeStruct((N, 1), jnp.int32),
    )(cn, ct)
    return out[:, 0]


def _onehot_body(lab_ref, o_ref):
    idx = lab_ref[...]
    ii = jax.lax.broadcasted_iota(jnp.int32, (_CH, _K), 1)
    o_ref[...] = (ii == idx).astype(jnp.float32)[None]


def _onehot_out_pallas(labels2d, batch):
    N = labels2d.shape[0]
    return pl.pallas_call(
        _onehot_body,
        grid=(batch, N // _CH),
        in_specs=[
            pl.BlockSpec((_CH, 1), lambda b, i: (i, 0)),
        ],
        out_specs=pl.BlockSpec((1, _CH, _K), lambda b, i: (b, i, 0)),
        out_shape=jax.ShapeDtypeStruct((batch, N, _K), jnp.float32),
    )(labels2d)


def kernel(x, coords):
    N = coords.shape[0]
    c_mean = jnp.mean(coords, axis=0)
    c_std = jnp.std(coords, axis=0) + 1e-05
    cn = (coords - c_mean) / c_std
    ones = jnp.ones((N,), dtype=cn.dtype)
    cents = cn[:_K]
    labels = None
    for it in range(_ITERS):
        labels = _labels_pallas(cn, cents.T)
        if it < _ITERS - 1:
            sums = jax.ops.segment_sum(cn, labels, num_segments=_K)
            counts = jax.ops.segment_sum(ones, labels, num_segments=_K)
            cents = sums / jnp.maximum(counts, 1.0)[:, None]
    return _onehot_out_pallas(labels[:, None], x.shape[0])


# speedup vs baseline: 1.1365x; 1.1365x over previous
import jax
import jax.numpy as jnp
from jax.experimental import pallas as pl

_K = 128
_ITERS = 10
_CH = 2000


def _dist_idx(cn_ref, ct_ref):
    px = cn_ref[:, 0:1]
    py = cn_ref[:, 1:2]
    pz = cn_ref[:, 2:3]
    cx = ct_ref[0:1, :]
    cy = ct_ref[1:2, :]
    cz = ct_ref[2:3, :]
    dx = px - cx
    dy = py - cy
    dz = pz - cz
    d = (dx * dx + dy * dy) + dz * dz
    m = jnp.min(d, axis=1, keepdims=True)
    ii = jax.lax.broadcasted_iota(jnp.int32, d.shape, 1)
    idx = jnp.min(jnp.where(d == m, ii, jnp.int32(2147483647)), axis=1,
                  keepdims=True)
    return idx


def _argmin_body(cn_ref, ct_ref, lab_ref):
    lab_ref[...] = _dist_idx(cn_ref, ct_ref)


def _labels_pallas(cn, ct):
    N = cn.shape[0]
    out = pl.pallas_call(
        _argmin_body,
        grid=(N // _CH,),
        in_specs=[
            pl.BlockSpec((_CH, 3), lambda i: (i, 0)),
            pl.BlockSpec((3, _K), lambda i: (0, 0)),
        ],
        out_specs=pl.BlockSpec((_CH, 1), lambda i: (i, 0)),
        out_shape=jax.ShapeDtypeStruct((N, 1), jnp.int32),
    )(cn, ct)
    return out[:, 0]


def _onehot_body(lab_ref, o_ref):
    idx = lab_ref[...]
    ii = jax.lax.broadcasted_iota(jnp.int32, (_CH, _K), 1)
    o_ref[...] = (ii == idx).astype(jnp.float32)[None]


def _onehot_out_pallas(labels2d, batch):
    N = labels2d.shape[0]
    return pl.pallas_call(
        _onehot_body,
        grid=(batch, N // _CH),
        in_specs=[
            pl.BlockSpec((_CH, 1), lambda b, i: (i, 0)),
        ],
        out_specs=pl.BlockSpec((1, _CH, _K), lambda b, i: (b, i, 0)),
        out_shape=jax.ShapeDtypeStruct((batch, N, _K), jnp.float32),
    )(labels2d)


def kernel(x, coords):
    N = coords.shape[0]
    c_mean = jnp.mean(coords, axis=0)
    c_std = jnp.std(coords, axis=0) + 1e-05
    cn = (coords - c_mean) / c_std
    ones = jnp.ones((N,), dtype=cn.dtype)
    cents = cn[:_K]
    iota = jax.lax.iota(jnp.int32, N)
    labels = None
    for it in range(_ITERS):
        labels = _labels_pallas(cn, cents.T)
        if it < _ITERS - 1:
            sl, si = jax.lax.sort((labels, iota), dimension=0, num_keys=1,
                                  is_stable=True)
            upd = jnp.take(cn, si, axis=0)
            sums = jax.lax.scatter_add(
                jnp.zeros((_K, 3), cn.dtype), sl[:, None], upd,
                jax.lax.ScatterDimensionNumbers(
                    update_window_dims=(1,), inserted_window_dims=(0,),
                    scatter_dims_to_operand_dims=(0,)),
                indices_are_sorted=True, unique_indices=False)
            counts = (jnp.searchsorted(sl, jnp.arange(1, _K + 1))
                      - jnp.searchsorted(sl, jnp.arange(_K))).astype(cn.dtype)
            cents = sums / jnp.maximum(counts, 1.0)[:, None]
    return _onehot_out_pallas(labels[:, None], x.shape[0])
